# baseline (speedup 1.0000x reference)
import jax
import jax.numpy as jnp
from jax.experimental import pallas as pl
from jax.experimental.pallas import tpu as pltpu

_NB_KNN = (10, 20, 100, 200)
_MAX_K = 200
_T = 0.07
_NUM_CLASSES = 1000
_CHUNK = 2048


def _matmul_body(q_ref, kt_ref, out_ref):
    out_ref[...] = jnp.dot(q_ref[...], kt_ref[...],
                           preferred_element_type=jnp.float32)


def kernel(features_rank, train_features, train_labels):
    q, d = features_rank.shape
    k, _ = train_features.shape
    k_pad = ((k + _CHUNK - 1) // _CHUNK) * _CHUNK
    kt = jnp.pad(train_features.T, ((0, 0), (0, k_pad - k)))

    n_chunks = k_pad // _CHUNK
    sims = pl.pallas_call(
        _matmul_body,
        grid=(n_chunks,),
        in_specs=[
            pl.BlockSpec((q, d), lambda i: (0, 0)),
            pl.BlockSpec((d, _CHUNK), lambda i: (0, i)),
        ],
        out_specs=pl.BlockSpec((q, _CHUNK), lambda i: (0, i)),
        out_shape=jax.ShapeDtypeStruct((q, k_pad), jnp.float32),
    )(features_rank, kt)

    sims = sims[:, :k]
    topk_sims, indices = jax.lax.top_k(sims, _MAX_K)
    neighbors_labels = jnp.take(train_labels, indices)
    w = jax.nn.softmax(topk_sims / _T, axis=1)
    onehot = jax.nn.one_hot(neighbors_labels, _NUM_CLASSES, dtype=w.dtype)
    matmul = onehot * w[:, :, None]
    return tuple(jnp.sum(matmul[:, :kk, :], axis=1) for kk in _NB_KNN)
